# baseline (speedup 1.0000x reference)
import jax
import jax.numpy as jnp
from jax import lax
from jax.experimental import pallas as pl
from jax.experimental.pallas import tpu as pltpu, tpu_sc as plsc

B, S, C = 32, 8192, 128
K = 4
L = 16
NG = C // L

NB_SC = 8
NB_TC = B - NB_SC
NSH = 32 // NB_SC
SROWS = S // NSH
CS = 256
NPAIR = SROWS // (2 * CS)

TC_BB = 4


def _merge_sorted4(st, e):
    s1, s2, s3, s4 = st
    e0, e1, e2, e3 = e
    b0 = jnp.maximum(s1, e0)
    b1 = jnp.maximum(s2, e1)
    b2 = jnp.maximum(s3, e2)
    b3 = jnp.maximum(s4, e3)
    u0 = jnp.maximum(b0, b2)
    u2 = jnp.minimum(b0, b2)
    u1 = jnp.maximum(b1, b3)
    u3 = jnp.minimum(b1, b3)
    return [jnp.maximum(u0, u1), jnp.minimum(u0, u1),
            jnp.maximum(u2, u3), jnp.minimum(u2, u3)]


def _sort4_asc(v0, v1, v2, v3):
    a = jnp.minimum(v0, v1)
    b = jnp.maximum(v0, v1)
    c = jnp.minimum(v2, v3)
    d = jnp.maximum(v2, v3)
    e0 = jnp.minimum(a, c)
    t1 = jnp.maximum(a, c)
    e3 = jnp.maximum(b, d)
    t2 = jnp.minimum(b, d)
    e1 = jnp.minimum(t1, t2)
    e2 = jnp.maximum(t1, t2)
    return [e0, e1, e2, e3]


def _sc_insert_rows(buf, st):
    def row_body(r, st):
        st = list(st)
        for g in range(NG):
            v0 = buf[4 * r, pl.ds(g * L, L)]
            v1 = buf[4 * r + 1, pl.ds(g * L, L)]
            v2 = buf[4 * r + 2, pl.ds(g * L, L)]
            v3 = buf[4 * r + 3, pl.ds(g * L, L)]
            e = _sort4_asc(v0, v1, v2, v3)
            st[4 * g:4 * g + 4] = _merge_sorted4(st[4 * g:4 * g + 4], e)
        return tuple(st)

    return lax.fori_loop(0, CS // 4, row_body, st)


def _sc_body(x_hbm, cand_hbm, buf0, buf1, obuf, sem0, sem1):
    cid = lax.axis_index("c")
    sid = lax.axis_index("s")
    wid = sid * 2 + cid
    batch = wid % NB_SC
    shard = wid // NB_SC
    base = shard * SROWS

    pltpu.async_copy(x_hbm.at[batch, pl.ds(base, CS)], buf0, sem0)
    pltpu.async_copy(x_hbm.at[batch, pl.ds(base + CS, CS)], buf1, sem1)

    neg = jnp.full((L,), -jnp.inf, dtype=jnp.float32)

    def pair_body(i, st):
        off = base + 2 * i * CS
        pltpu.make_async_copy(x_hbm.at[batch, pl.ds(0, CS)], buf0, sem0).wait()
        st = _sc_insert_rows(buf0, st)

        @pl.when(i < NPAIR - 1)
        def _():
            pltpu.async_copy(x_hbm.at[batch, pl.ds(off + 2 * CS, CS)],
                             buf0, sem0)

        pltpu.make_async_copy(x_hbm.at[batch, pl.ds(0, CS)], buf1, sem1).wait()
        st = _sc_insert_rows(buf1, st)

        @pl.when(i < NPAIR - 1)
        def _():
            pltpu.async_copy(x_hbm.at[batch, pl.ds(off + 3 * CS, CS)],
                             buf1, sem1)

        return st

    init = tuple(neg for _ in range(NG * K))
    st = lax.fori_loop(0, NPAIR, pair_body, init)

    for g in range(NG):
        for j in range(K):
            obuf[j, pl.ds(g * L, L)] = st[4 * g + j]
    pltpu.sync_copy(obuf, cand_hbm.at[shard, batch])


def _sc_candidates(x):
    mesh = plsc.VectorSubcoreMesh(core_axis_name="c", subcore_axis_name="s")
    kfn = pl.kernel(
        _sc_body,
        out_type=jax.ShapeDtypeStruct((NSH, NB_SC, K, C), jnp.float32),
        mesh=mesh,
        scratch_types=[
            pltpu.VMEM((CS, C), jnp.float32),
            pltpu.VMEM((CS, C), jnp.float32),
            pltpu.VMEM((K, C), jnp.float32),
            pltpu.SemaphoreType.DMA,
            pltpu.SemaphoreType.DMA,
        ],
    )
    return kfn(x)


def _merge_desc(a, b):
    return _merge_sorted4(a, [b[3], b[2], b[1], b[0]])


def _tc_body(x_ref, o_ref):
    sub = lax.broadcasted_iota(jnp.int32, (K * 8, C), 0)
    for bb in range(TC_BB):
        lists = []
        for r in range(S // 32):
            e = _sort4_asc(x_ref[bb, 32 * r:32 * r + 8, :],
                           x_ref[bb, 32 * r + 8:32 * r + 16, :],
                           x_ref[bb, 32 * r + 16:32 * r + 24, :],
                           x_ref[bb, 32 * r + 24:32 * r + 32, :])
            lists.append([e[3], e[2], e[1], e[0]])
        while len(lists) > 1:
            lists = [_merge_desc(lists[i], lists[i + 1])
                     for i in range(0, len(lists), 2)]
        vals = jnp.concatenate(lists[0], axis=0)
        outs = []
        for _ in range(K):
            m = jnp.max(vals, axis=0, keepdims=True)
            eq = vals == m
            first = jnp.min(jnp.where(eq, sub, K * 8), axis=0, keepdims=True)
            vals = jnp.where(sub == first, -jnp.inf, vals)
            outs.append(m)
        o_ref[bb] = jnp.concatenate(outs, axis=0)


def _tc_topk(x):
    return pl.pallas_call(
        _tc_body,
        grid=(NB_TC // TC_BB,),
        in_specs=[pl.BlockSpec((TC_BB, S, C),
                               lambda b: (b + NB_SC // TC_BB, 0, 0))],
        out_specs=pl.BlockSpec((TC_BB, K, C), lambda b: (b, 0, 0)),
        out_shape=jax.ShapeDtypeStruct((NB_TC, K, C), jnp.float32),
        compiler_params=pltpu.CompilerParams(
            dimension_semantics=("arbitrary",)),
    )(x)


def _merge_body(cand_ref, tc_ref, o_ref):
    st = [cand_ref[0, :, j, :] for j in range(K)]
    for sh in range(1, NSH):
        e = [cand_ref[sh, :, K - 1 - j, :] for j in range(K)]
        st = _merge_sorted4(st, e)
    sc_blk = jnp.stack(st, axis=1)
    full = jnp.concatenate([sc_blk, tc_ref[...]], axis=0)
    o_ref[...] = jnp.transpose(full, (0, 2, 1))


def _merge_candidates(cand, out_tc):
    return pl.pallas_call(
        _merge_body,
        out_shape=jax.ShapeDtypeStruct((B, C, K), jnp.float32),
    )(cand, out_tc)


def kernel(inputs):
    cand = _sc_candidates(inputs)
    out_tc = _tc_topk(inputs)
    out_ck = _merge_candidates(cand, out_tc)
    return out_ck.reshape(B, C * K)


# speedup vs baseline: 1.0545x; 1.0545x over previous
import jax
import jax.numpy as jnp
from jax import lax
from jax.experimental import pallas as pl
from jax.experimental.pallas import tpu as pltpu, tpu_sc as plsc

B, S, C = 32, 8192, 128
K = 4
L = 16
NG = C // L

NB_SC = 8
NB_TC = B - NB_SC
NSH = 32 // NB_SC
SROWS = S // NSH
CS = 256
NPAIR = SROWS // (2 * CS)

TC_BB = 4


def _merge_sorted4(st, e):
    s1, s2, s3, s4 = st
    e0, e1, e2, e3 = e
    b0 = jnp.maximum(s1, e0)
    b1 = jnp.maximum(s2, e1)
    b2 = jnp.maximum(s3, e2)
    b3 = jnp.maximum(s4, e3)
    u0 = jnp.maximum(b0, b2)
    u2 = jnp.minimum(b0, b2)
    u1 = jnp.maximum(b1, b3)
    u3 = jnp.minimum(b1, b3)
    return [jnp.maximum(u0, u1), jnp.minimum(u0, u1),
            jnp.maximum(u2, u3), jnp.minimum(u2, u3)]


def _sort4_asc(v0, v1, v2, v3):
    a = jnp.minimum(v0, v1)
    b = jnp.maximum(v0, v1)
    c = jnp.minimum(v2, v3)
    d = jnp.maximum(v2, v3)
    e0 = jnp.minimum(a, c)
    t1 = jnp.maximum(a, c)
    e3 = jnp.maximum(b, d)
    t2 = jnp.minimum(b, d)
    e1 = jnp.minimum(t1, t2)
    e2 = jnp.maximum(t1, t2)
    return [e0, e1, e2, e3]


def _sc_insert_rows(buf, st):
    def row_body(r, st):
        st = list(st)
        for g in range(NG):
            v0 = buf[4 * r, pl.ds(g * L, L)]
            v1 = buf[4 * r + 1, pl.ds(g * L, L)]
            v2 = buf[4 * r + 2, pl.ds(g * L, L)]
            v3 = buf[4 * r + 3, pl.ds(g * L, L)]
            e = _sort4_asc(v0, v1, v2, v3)
            st[4 * g:4 * g + 4] = _merge_sorted4(st[4 * g:4 * g + 4], e)
        return tuple(st)

    return lax.fori_loop(0, CS // 4, row_body, st)


def _sc_body(x_hbm, cand_hbm, buf0, buf1, obuf, sem0, sem1):
    cid = lax.axis_index("c")
    sid = lax.axis_index("s")
    wid = sid * 2 + cid
    batch = wid % NB_SC
    shard = wid // NB_SC
    base = shard * SROWS

    pltpu.async_copy(x_hbm.at[batch, pl.ds(base, CS)], buf0, sem0)
    pltpu.async_copy(x_hbm.at[batch, pl.ds(base + CS, CS)], buf1, sem1)

    neg = jnp.full((L,), -jnp.inf, dtype=jnp.float32)

    def pair_body(i, st):
        off = base + 2 * i * CS
        pltpu.make_async_copy(x_hbm.at[batch, pl.ds(0, CS)], buf0, sem0).wait()
        st = _sc_insert_rows(buf0, st)

        @pl.when(i < NPAIR - 1)
        def _():
            pltpu.async_copy(x_hbm.at[batch, pl.ds(off + 2 * CS, CS)],
                             buf0, sem0)

        pltpu.make_async_copy(x_hbm.at[batch, pl.ds(0, CS)], buf1, sem1).wait()
        st = _sc_insert_rows(buf1, st)

        @pl.when(i < NPAIR - 1)
        def _():
            pltpu.async_copy(x_hbm.at[batch, pl.ds(off + 3 * CS, CS)],
                             buf1, sem1)

        return st

    init = tuple(neg for _ in range(NG * K))
    st = lax.fori_loop(0, NPAIR, pair_body, init)

    for g in range(NG):
        for j in range(K):
            obuf[j, pl.ds(g * L, L)] = st[4 * g + j]
    pltpu.sync_copy(obuf, cand_hbm.at[shard, batch])


def _sc_candidates(x):
    mesh = plsc.VectorSubcoreMesh(core_axis_name="c", subcore_axis_name="s")
    kfn = pl.kernel(
        _sc_body,
        out_type=jax.ShapeDtypeStruct((NSH, NB_SC, K, C), jnp.float32),
        mesh=mesh,
        scratch_types=[
            pltpu.VMEM((CS, C), jnp.float32),
            pltpu.VMEM((CS, C), jnp.float32),
            pltpu.VMEM((K, C), jnp.float32),
            pltpu.SemaphoreType.DMA,
            pltpu.SemaphoreType.DMA,
        ],
    )
    return kfn(x)


def _merge_desc(a, b):
    return _merge_sorted4(a, [b[3], b[2], b[1], b[0]])


def _tc_body(x_ref, o_ref):
    sub = lax.broadcasted_iota(jnp.int32, (K * 8, C), 0)
    for bb in range(TC_BB):
        lists = []
        for r in range(S // 32):
            e = _sort4_asc(x_ref[bb, 32 * r:32 * r + 8, :],
                           x_ref[bb, 32 * r + 8:32 * r + 16, :],
                           x_ref[bb, 32 * r + 16:32 * r + 24, :],
                           x_ref[bb, 32 * r + 24:32 * r + 32, :])
            lists.append([e[3], e[2], e[1], e[0]])
        while len(lists) > 1:
            lists = [_merge_desc(lists[i], lists[i + 1])
                     for i in range(0, len(lists), 2)]
        vals = jnp.concatenate(lists[0], axis=0)
        outs = []
        for _ in range(K):
            m = jnp.max(vals, axis=0, keepdims=True)
            eq = vals == m
            first = jnp.min(jnp.where(eq, sub, K * 8), axis=0, keepdims=True)
            vals = jnp.where(sub == first, -jnp.inf, vals)
            outs.append(m)
        o_ref[bb] = jnp.concatenate(outs, axis=0)


def _tc_topk(x):
    return pl.pallas_call(
        _tc_body,
        grid=(NB_TC // TC_BB,),
        in_specs=[pl.BlockSpec((TC_BB, S, C),
                               lambda b: (b + NB_SC // TC_BB, 0, 0))],
        out_specs=pl.BlockSpec((TC_BB, K, C), lambda b: (b, 0, 0)),
        out_shape=jax.ShapeDtypeStruct((NB_TC, K, C), jnp.float32),
        compiler_params=pltpu.CompilerParams(
            dimension_semantics=("arbitrary",)),
    )(x)


def _merge_body(cand_ref, tc_ref, o_ref):
    st = [cand_ref[0, :, j, :] for j in range(K)]
    for sh in range(1, NSH):
        e = [cand_ref[sh, :, K - 1 - j, :] for j in range(K)]
        st = _merge_sorted4(st, e)
    sc_blk = jnp.stack(st, axis=1)
    full = jnp.concatenate([sc_blk, tc_ref[...]], axis=0)
    ck = jnp.transpose(full, (0, 2, 1))
    o_ref[...] = ck.reshape(B, C * K)


def _merge_candidates(cand, out_tc):
    return pl.pallas_call(
        _merge_body,
        out_shape=jax.ShapeDtypeStruct((B, C * K), jnp.float32),
    )(cand, out_tc)


def kernel(inputs):
    cand = _sc_candidates(inputs)
    out_tc = _tc_topk(inputs)
    return _merge_candidates(cand, out_tc)
